# baseline (speedup 1.0000x reference)
import functools

import jax
import jax.numpy as jnp
from jax import lax
from jax.experimental import pallas as pl
from jax.experimental.pallas import tpu as pltpu
from jax.experimental.pallas import tpu_sc as plsc

EMBED = 64
BATCH = 4096
SEQ = 200
HALF = BATCH // 2
B = BATCH * SEQ

NSLICE = 4
SEQ_SL = SEQ // NSLICE
ROWS_SL = SEQ_SL * BATCH

NC = 2
NS = 16
NW = NC * NS
CHUNK = 512
HC = CHUNK // 2
N_CHUNKS_SL = ROWS_SL // CHUNK


@functools.lru_cache(maxsize=1)
def _make_sc_gather():
    mesh = plsc.VectorSubcoreMesh(core_axis_name="c", subcore_axis_name="s")

    @functools.partial(
        pl.kernel,
        mesh=mesh,
        out_type=jax.ShapeDtypeStruct((ROWS_SL, EMBED), jnp.float32),
        scratch_types=[
            pltpu.VMEM((CHUNK,), jnp.int32),
            pltpu.VMEM((CHUNK, EMBED), jnp.float32),
            pltpu.SemaphoreType.DMA,
        ],
        compiler_params=pltpu.CompilerParams(use_tc_tiling_on_sc=False),
    )
    def _sc_gather(tok_hbm, table_hbm, out_hbm, idx_v, rows_v, sem):
        wid = lax.axis_index("s") * NC + lax.axis_index("c")
        n_w = (N_CHUNKS_SL + NW - 1 - wid) // NW
        def body(i, carry):
            off = (wid + i * NW) * CHUNK
            pltpu.sync_copy(tok_hbm.at[pl.ds(off, CHUNK)], idx_v)
            pltpu.async_copy(table_hbm.at[idx_v], rows_v, sem).wait()
            pltpu.sync_copy(rows_v, out_hbm.at[pl.ds(off, CHUNK)])
            return carry

        lax.fori_loop(0, n_w, body, 0)

    return _sc_gather


S_BLK = 5


def _ln_t_body(y_ref, pos_ref, gamma_ref, beta_ref, *refs):
    out_ref = refs[-1]
    g = gamma_ref[...]
    bta = beta_ref[...]
    for si in range(S_BLK):
        x = y_ref[si * HALF:(si + 1) * HALF, :] + pos_ref[si, 0]
        for h in (0, 1):
            t = x[:, h * EMBED:(h + 1) * EMBED].T
            mean = jnp.mean(t, axis=0, keepdims=True)
            tc = t - mean
            var = jnp.mean(tc * tc, axis=0, keepdims=True)
            yh = tc * lax.rsqrt(var + 1e-5) * g + bta
            out_ref[si, :, h * HALF:(h + 1) * HALF] = yh


def _ln_pallas(k, y, pos128, g64, b64, prev=None, interpret=False):
    blk0 = k * (SEQ_SL // S_BLK)
    in_specs = [
        pl.BlockSpec((S_BLK * HALF, 2 * EMBED), lambda i: (i, 0)),
        pl.BlockSpec((S_BLK, 1, 2 * EMBED), lambda i: (i, 0, 0)),
        pl.BlockSpec((EMBED, 1), lambda i: (0, 0)),
        pl.BlockSpec((EMBED, 1), lambda i: (0, 0)),
    ]
    args = [y, pos128, g64, b64]
    aliases = {}
    if prev is not None:
        in_specs.append(pl.BlockSpec(memory_space=pl.ANY))
        args.append(prev)
        aliases = {4: 0}
    return pl.pallas_call(
        _ln_t_body,
        grid=(SEQ_SL // S_BLK,),
        in_specs=in_specs,
        out_specs=pl.BlockSpec((S_BLK, EMBED, BATCH),
                               lambda i: (blk0 + i, 0, 0)),
        out_shape=jax.ShapeDtypeStruct((SEQ, EMBED, BATCH), jnp.float32),
        input_output_aliases=aliases,
        interpret=interpret,
    )(*args)


def kernel(input_tokens, token_table, position_table, ln_gamma, ln_beta):
    tok_sm = (
        input_tokens.T.astype(jnp.int32)
        .reshape(SEQ, 2, HALF)
        .transpose(0, 2, 1)
        .reshape(B)
    )
    pos128 = jnp.concatenate([position_table, position_table], axis=1).reshape(
        SEQ, 1, 2 * EMBED
    )
    g64 = ln_gamma.reshape(EMBED, 1)
    b64 = ln_beta.reshape(EMBED, 1)

    sc_gather = _make_sc_gather()
    gathered = [
        sc_gather(tok_sm[k * ROWS_SL:(k + 1) * ROWS_SL], token_table)
        for k in range(NSLICE)
    ]
    out3 = None
    for k in range(NSLICE):
        y = gathered[k].reshape(ROWS_SL // 2, 2 * EMBED)
        pos_k = pos128[k * SEQ_SL:(k + 1) * SEQ_SL]
        out3 = _ln_pallas(k, y, pos_k, g64, b64, prev=out3)
    return out3.transpose(2, 0, 1)
